# baseline (speedup 1.0000x reference)
import functools

import jax
import jax.numpy as jnp
from jax import lax
from jax.experimental import pallas as pl
from jax.experimental.pallas import tpu as pltpu
from jax.experimental.pallas import tpu_sc as plsc

N = 10000
D = 256
DQ = 64
E = 160000
NPAD = 10240
DUMMY = N
NSUB = 16
NCORE = 2
NW = NCORE * NSUB
K = 128
NCH = 80
EPAD = NSUB * NCH * K
NB = 8
ROWS_PER_TILE = NPAD // NSUB

_mesh = plsc.VectorSubcoreMesh(core_axis_name="c", subcore_axis_name="s")


@functools.partial(
    pl.kernel,
    out_type=jax.ShapeDtypeStruct((NW, NPAD), jnp.float32),
    mesh=_mesh,
    compiler_params=pltpu.CompilerParams(needs_layout_passes=False),
    scratch_types=[
        pltpu.VMEM((NCH, K), jnp.int32),
        pltpu.VMEM((NPAD,), jnp.float32),
    ],
)
def _deg(dst_hbm, out_hbm, dstv, ldeg):
    c = lax.axis_index("c")
    s = lax.axis_index("s")
    pltpu.sync_copy(dst_hbm.at[s], dstv)
    zero16 = jnp.zeros((16,), jnp.float32)
    ones16 = jnp.ones((16,), jnp.float32)

    def zbody(i, _):
        ldeg[pl.ds(i * 16, 16)] = zero16
        return 0

    lax.fori_loop(0, NPAD // 16, zbody, 0)

    half = NCH // 2

    def ebody(j, _):
        jj = c * half + j
        for q in range(K // 16):
            idx = dstv[jj, pl.ds(q * 16, 16)]
            plsc.addupdate_scatter(ldeg, [idx], ones16)
        return 0

    lax.fori_loop(0, half, ebody, 0)
    pltpu.sync_copy(ldeg, out_hbm.at[s * NCORE + c])


_QSD = jax.ShapeDtypeStruct((NPAD, DQ), jnp.float32)


@functools.partial(
    pl.kernel,
    out_type=(_QSD, _QSD, _QSD, _QSD),
    mesh=_mesh,
    compiler_params=pltpu.CompilerParams(
        needs_layout_passes=False, use_tc_tiling_on_sc=False
    ),
    scratch_types=[
        pltpu.VMEM((NCH, K), jnp.int32),
        pltpu.VMEM((NCH, K), jnp.int32),
        pltpu.VMEM((NB, K, DQ), jnp.float32),
        pltpu.VMEM_SHARED((NPAD, DQ), jnp.float32),
        pltpu.SemaphoreType.DMA((NB,)),
        pltpu.SemaphoreType.DMA((NB,)),
    ],
)
def _prop(src_hbm, dst_hbm, g0, g1, g2, g3,
          o0, o1, o2, o3, srcv, dstv, bufs, accum, gsem, ssem):
    c = lax.axis_index("c")
    s = lax.axis_index("s")
    pltpu.sync_copy(src_hbm.at[s], srcv)
    pltpu.sync_copy(dst_hbm.at[s], dstv)

    gq = (g0, g1, g2, g3)
    oq = (o0, o1, o2, o3)

    for p in range(2):
        for t in range(ROWS_PER_TILE // K):
            blk = pl.ds(s * ROWS_PER_TILE + t * K, K)

            @pl.when(c == 0)
            def _():
                pltpu.sync_copy(gq[p].at[blk], accum.at[blk])

            @pl.when(c == 1)
            def _():
                pltpu.sync_copy(gq[2 + p].at[blk], accum.at[blk])

        plsc.subcore_barrier()

        def gissue(j, b):
            @pl.when(c == 0)
            def _():
                pltpu.async_copy(gq[p].at[srcv.at[j]], bufs.at[b], gsem.at[b])

            @pl.when(c == 1)
            def _():
                pltpu.async_copy(
                    gq[2 + p].at[srcv.at[j]], bufs.at[b], gsem.at[b]
                )

        for j in range(NB):
            gissue(j, j)

        def mb(j, _):
            b = lax.rem(j, NB)
            pltpu.make_async_copy(
                g0.at[pl.ds(0, K)], bufs.at[b], gsem.at[b]
            ).wait()
            pltpu.async_copy(
                bufs.at[b], accum.at[dstv.at[j]], ssem.at[b], add=True
            )
            jn = j + NB

            @pl.when(jn < NCH)
            def _():
                pltpu.make_async_copy(
                    g0.at[pl.ds(0, K)], bufs.at[b], ssem.at[b]
                ).wait()
                gissue(jn, b)

            return 0

        lax.fori_loop(0, NCH, mb, 0)
        for b in range(NB):
            pltpu.make_async_copy(
                g0.at[pl.ds(0, K)], bufs.at[b], ssem.at[b]
            ).wait()
        plsc.subcore_barrier()
        rows = pl.ds(s * ROWS_PER_TILE, ROWS_PER_TILE)

        @pl.when(c == 0)
        def _():
            pltpu.sync_copy(accum.at[rows], oq[p].at[rows])

        @pl.when(c == 1)
        def _():
            pltpu.sync_copy(accum.at[rows], oq[2 + p].at[rows])


RB = 1024


def _mm_body(deg_ref, x_ref, w_ref, g0_ref, g1_ref, g2_ref, g3_ref):
    deg = 1.0 + jnp.sum(deg_ref[...], axis=0)
    dinv = lax.rsqrt(deg)
    h = jnp.dot(x_ref[...], w_ref[...], preferred_element_type=jnp.float32)
    g = h * dinv[:, None]
    g0_ref[...] = g[:, 0 * DQ:1 * DQ]
    g1_ref[...] = g[:, 1 * DQ:2 * DQ]
    g2_ref[...] = g[:, 2 * DQ:3 * DQ]
    g3_ref[...] = g[:, 3 * DQ:4 * DQ]


def _mm(deg, x, w):
    nb = NPAD // RB
    qspec = pl.BlockSpec((RB, DQ), lambda i: (i, 0))
    return pl.pallas_call(
        _mm_body,
        grid=(nb,),
        in_specs=[
            pl.BlockSpec((NW, RB), lambda i: (0, i)),
            pl.BlockSpec((RB, D), lambda i: (i, 0)),
            pl.BlockSpec((D, D), lambda i: (0, 0)),
        ],
        out_specs=(qspec, qspec, qspec, qspec),
        out_shape=(_QSD, _QSD, _QSD, _QSD),
    )(deg, x, w)


def _fin_body(q0_ref, q1_ref, q2_ref, q3_ref, deg_ref, b_ref, o_ref):
    deg = 1.0 + jnp.sum(deg_ref[...], axis=0)
    dinv = lax.rsqrt(deg)
    merged = jnp.concatenate(
        [q0_ref[...], q1_ref[...], q2_ref[...], q3_ref[...]], axis=1
    )
    o_ref[...] = merged * dinv[:, None] + b_ref[...][None, :]


def _fin(s0, s1, s2, s3, deg, b):
    nb = NPAD // RB
    qspec = pl.BlockSpec((RB, DQ), lambda i: (i, 0))
    return pl.pallas_call(
        _fin_body,
        grid=(nb,),
        in_specs=[
            qspec,
            qspec,
            qspec,
            qspec,
            pl.BlockSpec((NW, RB), lambda i: (0, i)),
            pl.BlockSpec((D,), lambda i: (0,)),
        ],
        out_specs=pl.BlockSpec((RB, D), lambda i: (i, 0)),
        out_shape=jax.ShapeDtypeStruct((N, D), jnp.float32),
    )(s0, s1, s2, s3, deg, b)


def kernel(x, edge_index, W, b):
    src = edge_index[0].astype(jnp.int32)
    dst = edge_index[1].astype(jnp.int32)
    npad_e = EPAD - E
    srcp = jnp.concatenate([src, jnp.zeros((npad_e,), jnp.int32)])
    dstp = jnp.concatenate([dst, jnp.full((npad_e,), DUMMY, jnp.int32)])
    srcr = srcp.reshape(NSUB, NCH, K)
    dstr = dstp.reshape(NSUB, NCH, K)

    deg = _deg(dstr)
    g0, g1, g2, g3 = _mm(deg, x, W)
    s0, s1, s2, s3 = _prop(srcr, dstr, g0, g1, g2, g3)
    return _fin(s0, s1, s2, s3, deg, b)


# speedup vs baseline: 1.3920x; 1.3920x over previous
import functools

import jax
import jax.numpy as jnp
from jax import lax
from jax.experimental import pallas as pl
from jax.experimental.pallas import tpu as pltpu
from jax.experimental.pallas import tpu_sc as plsc

N = 10000
D = 256
DQ = 64
E = 160000
NPAD = 10240
DUMMY = N
NSUB = 16
NCORE = 2
NW = NCORE * NSUB
K = 128
NCH = 84
EPAD = NSUB * NCH * K
NB = 8
ROWS_PER_TILE = NPAD // NSUB

_mesh = plsc.VectorSubcoreMesh(core_axis_name="c", subcore_axis_name="s")


@functools.partial(
    pl.kernel,
    out_type=jax.ShapeDtypeStruct((NW, NPAD), jnp.float32),
    mesh=_mesh,
    compiler_params=pltpu.CompilerParams(needs_layout_passes=False),
    scratch_types=[
        pltpu.VMEM((NCH, K), jnp.int32),
        pltpu.VMEM((NPAD,), jnp.float32),
    ],
)
def _deg(dst_hbm, out_hbm, dstv, ldeg):
    c = lax.axis_index("c")
    s = lax.axis_index("s")
    pltpu.sync_copy(dst_hbm.at[s], dstv)
    zero16 = jnp.zeros((16,), jnp.float32)
    ones16 = jnp.ones((16,), jnp.float32)

    def zbody(i, _):
        ldeg[pl.ds(i * 16, 16)] = zero16
        return 0

    lax.fori_loop(0, NPAD // 16, zbody, 0)

    half = NCH // 2

    def ebody(j, _):
        jj = c * half + j
        for q in range(K // 16):
            idx = dstv[jj, pl.ds(q * 16, 16)]
            plsc.addupdate_scatter(ldeg, [idx], ones16)
        return 0

    lax.fori_loop(0, half, ebody, 0)
    pltpu.sync_copy(ldeg, out_hbm.at[s * NCORE + c])


_QSD = jax.ShapeDtypeStruct((NPAD, DQ), jnp.float32)


@functools.partial(
    pl.kernel,
    out_type=(_QSD, _QSD, _QSD, _QSD),
    mesh=_mesh,
    compiler_params=pltpu.CompilerParams(
        needs_layout_passes=False, use_tc_tiling_on_sc=False
    ),
    scratch_types=[
        pltpu.VMEM((NCH, K), jnp.int32),
        pltpu.VMEM((NCH, K), jnp.int32),
        pltpu.VMEM((NB, K, DQ), jnp.float32),
        pltpu.VMEM_SHARED((NPAD, DQ), jnp.float32),
        pltpu.SemaphoreType.DMA((NB,)),
        pltpu.SemaphoreType.DMA((NB,)),
    ],
)
def _prop(src_hbm, dst_hbm, g0, g1, g2, g3,
          o0, o1, o2, o3, srcv, dstv, bufs, accum, gsem, ssem):
    c = lax.axis_index("c")
    s = lax.axis_index("s")
    pltpu.sync_copy(src_hbm.at[s], srcv)
    pltpu.sync_copy(dst_hbm.at[s], dstv)

    zero16 = jnp.zeros((16,), jnp.float32)
    gq = (g0, g1, g2, g3)
    oq = (o0, o1, o2, o3)

    for p in range(2):
        def zb(i, _):
            for q in range(DQ // 16):
                bufs[0, i, pl.ds(q * 16, 16)] = zero16
            return 0

        lax.fori_loop(0, K, zb, 0)
        for t in range(ROWS_PER_TILE // K):
            pltpu.sync_copy(
                bufs.at[0], accum.at[pl.ds(s * ROWS_PER_TILE + t * K, K)]
            )
        plsc.subcore_barrier()

        def gissue(j, b):
            @pl.when(c == 0)
            def _():
                pltpu.async_copy(gq[p].at[srcv.at[j]], bufs.at[b], gsem.at[b])

            @pl.when(c == 1)
            def _():
                pltpu.async_copy(
                    gq[2 + p].at[srcv.at[j]], bufs.at[b], gsem.at[b]
                )

        for j in range(NB):
            gissue(j, j)

        def mb(j, _):
            b = lax.rem(j, NB)
            pltpu.make_async_copy(
                g0.at[pl.ds(0, K)], bufs.at[b], gsem.at[b]
            ).wait()
            pltpu.async_copy(
                bufs.at[b], accum.at[dstv.at[j]], ssem.at[b], add=True
            )
            jn = j + NB

            @pl.when(jn < NCH)
            def _():
                pltpu.make_async_copy(
                    g0.at[pl.ds(0, K)], bufs.at[b], ssem.at[b]
                ).wait()
                gissue(jn, b)

            return 0

        lax.fori_loop(0, NCH, mb, 0)
        for b in range(NB):
            pltpu.make_async_copy(
                g0.at[pl.ds(0, K)], bufs.at[b], ssem.at[b]
            ).wait()
        plsc.subcore_barrier()
        rows = pl.ds(s * ROWS_PER_TILE, ROWS_PER_TILE)

        @pl.when(c == 0)
        def _():
            pltpu.sync_copy(accum.at[rows], oq[p].at[rows])

        @pl.when(c == 1)
        def _():
            pltpu.sync_copy(accum.at[rows], oq[2 + p].at[rows])


RB = 1024


def _mm_body(deg_ref, x_ref, w_ref, g0_ref, g1_ref, g2_ref, g3_ref):
    deg = jnp.sum(deg_ref[...], axis=0)
    dinv = lax.rsqrt(deg)
    h = jnp.dot(x_ref[...], w_ref[...], preferred_element_type=jnp.float32)
    g = h * dinv[:, None]
    g0_ref[...] = g[:, 0 * DQ:1 * DQ]
    g1_ref[...] = g[:, 1 * DQ:2 * DQ]
    g2_ref[...] = g[:, 2 * DQ:3 * DQ]
    g3_ref[...] = g[:, 3 * DQ:4 * DQ]


def _mm(deg, x, w):
    nb = NPAD // RB
    qspec = pl.BlockSpec((RB, DQ), lambda i: (i, 0))
    return pl.pallas_call(
        _mm_body,
        grid=(nb,),
        in_specs=[
            pl.BlockSpec((NW, RB), lambda i: (0, i)),
            pl.BlockSpec((RB, D), lambda i: (i, 0)),
            pl.BlockSpec((D, D), lambda i: (0, 0)),
        ],
        out_specs=(qspec, qspec, qspec, qspec),
        out_shape=(_QSD, _QSD, _QSD, _QSD),
    )(deg, x, w)


def _fin_body(q0_ref, q1_ref, q2_ref, q3_ref, deg_ref, b_ref, o_ref):
    deg = jnp.sum(deg_ref[...], axis=0)
    dinv = lax.rsqrt(deg)
    merged = jnp.concatenate(
        [q0_ref[...], q1_ref[...], q2_ref[...], q3_ref[...]], axis=1
    )
    o_ref[...] = merged * dinv[:, None] + b_ref[...][None, :]


def _fin(s0, s1, s2, s3, deg, b):
    nb = NPAD // RB
    qspec = pl.BlockSpec((RB, DQ), lambda i: (i, 0))
    return pl.pallas_call(
        _fin_body,
        grid=(nb,),
        in_specs=[
            qspec,
            qspec,
            qspec,
            qspec,
            pl.BlockSpec((NW, RB), lambda i: (0, i)),
            pl.BlockSpec((D,), lambda i: (0,)),
        ],
        out_specs=pl.BlockSpec((RB, D), lambda i: (i, 0)),
        out_shape=jax.ShapeDtypeStruct((N, D), jnp.float32),
    )(s0, s1, s2, s3, deg, b)


def kernel(x, edge_index, W, b):
    src = edge_index[0].astype(jnp.int32)
    dst = edge_index[1].astype(jnp.int32)
    loop = jnp.arange(N, dtype=jnp.int32)
    npad_e = EPAD - E - N
    srcp = jnp.concatenate([src, loop, jnp.zeros((npad_e,), jnp.int32)])
    dstp = jnp.concatenate([dst, loop, jnp.full((npad_e,), DUMMY, jnp.int32)])
    srcr = srcp.reshape(NSUB, NCH, K)
    dstr = dstp.reshape(NSUB, NCH, K)

    deg = _deg(dstr)
    g0, g1, g2, g3 = _mm(deg, x, W)
    s0, s1, s2, s3 = _prop(srcr, dstr, g0, g1, g2, g3)
    return _fin(s0, s1, s2, s3, deg, b)


# speedup vs baseline: 1.9370x; 1.3915x over previous
import functools

import jax
import jax.numpy as jnp
from jax import lax
from jax.experimental import pallas as pl
from jax.experimental.pallas import tpu as pltpu
from jax.experimental.pallas import tpu_sc as plsc

N = 10000
D = 256
DQ = 64
E = 160000
NPAD = 10240
DUMMY = N
NSUB = 16
NCORE = 2
NW = NCORE * NSUB
K = 128
NCH = 84
EPAD = NSUB * NCH * K
NB = 8
ROWS_PER_TILE = NPAD // NSUB

_mesh = plsc.VectorSubcoreMesh(core_axis_name="c", subcore_axis_name="s")


@functools.partial(
    pl.kernel,
    out_type=jax.ShapeDtypeStruct((NW, NPAD), jnp.float32),
    mesh=_mesh,
    compiler_params=pltpu.CompilerParams(needs_layout_passes=False),
    scratch_types=[
        pltpu.VMEM((NCH, K), jnp.int32),
        pltpu.VMEM((NPAD,), jnp.float32),
    ],
)
def _deg(dst_hbm, out_hbm, dstv, ldeg):
    c = lax.axis_index("c")
    s = lax.axis_index("s")
    pltpu.sync_copy(dst_hbm.at[s], dstv)
    zero16 = jnp.zeros((16,), jnp.float32)
    ones16 = jnp.ones((16,), jnp.float32)

    def zbody(i, _):
        ldeg[pl.ds(i * 16, 16)] = zero16
        return 0

    lax.fori_loop(0, NPAD // 16, zbody, 0)

    half = NCH // 2

    def ebody(j, _):
        jj = c * half + j
        for q in range(K // 16):
            idx = dstv[jj, pl.ds(q * 16, 16)]
            plsc.addupdate_scatter(ldeg, [idx], ones16)
        return 0

    lax.fori_loop(0, half, ebody, 0)
    pltpu.sync_copy(ldeg, out_hbm.at[s * NCORE + c])


_QSD = jax.ShapeDtypeStruct((NPAD, DQ), jnp.bfloat16)


@functools.partial(
    pl.kernel,
    out_type=(_QSD, _QSD, _QSD, _QSD),
    mesh=_mesh,
    compiler_params=pltpu.CompilerParams(
        needs_layout_passes=False, use_tc_tiling_on_sc=False
    ),
    scratch_types=[
        pltpu.VMEM((NCH, K), jnp.int32),
        pltpu.VMEM((NCH, K), jnp.int32),
        pltpu.VMEM((NB, K, DQ), jnp.bfloat16),
        pltpu.VMEM_SHARED((NPAD, DQ), jnp.bfloat16),
        pltpu.SemaphoreType.DMA((NB,)),
        pltpu.SemaphoreType.DMA((NB,)),
    ],
)
def _prop(src_hbm, dst_hbm, g0, g1, g2, g3,
          o0, o1, o2, o3, srcv, dstv, bufs, accum, gsem, ssem):
    c = lax.axis_index("c")
    s = lax.axis_index("s")
    pltpu.sync_copy(src_hbm.at[s], srcv)
    pltpu.sync_copy(dst_hbm.at[s], dstv)

    zero32 = jnp.zeros((32,), jnp.bfloat16)
    gq = (g0, g1, g2, g3)
    oq = (o0, o1, o2, o3)

    for p in range(2):
        def zb(i, _):
            for q in range(DQ // 32):
                bufs[0, i, pl.ds(q * 32, 32)] = zero32
            return 0

        lax.fori_loop(0, K, zb, 0)
        for t in range(ROWS_PER_TILE // K):
            pltpu.sync_copy(
                bufs.at[0], accum.at[pl.ds(s * ROWS_PER_TILE + t * K, K)]
            )
        plsc.subcore_barrier()

        def gissue(j, b):
            @pl.when(c == 0)
            def _():
                pltpu.async_copy(gq[p].at[srcv.at[j]], bufs.at[b], gsem.at[b])

            @pl.when(c == 1)
            def _():
                pltpu.async_copy(
                    gq[2 + p].at[srcv.at[j]], bufs.at[b], gsem.at[b]
                )

        for j in range(NB):
            gissue(j, j)

        def mb(j, _):
            b = lax.rem(j, NB)
            pltpu.make_async_copy(
                g0.at[pl.ds(0, K)], bufs.at[b], gsem.at[b]
            ).wait()
            pltpu.async_copy(
                bufs.at[b], accum.at[dstv.at[j]], ssem.at[b], add=True
            )
            jn = j + NB

            @pl.when(jn < NCH)
            def _():
                pltpu.make_async_copy(
                    g0.at[pl.ds(0, K)], bufs.at[b], ssem.at[b]
                ).wait()
                gissue(jn, b)

            return 0

        lax.fori_loop(0, NCH, mb, 0)
        for b in range(NB):
            pltpu.make_async_copy(
                g0.at[pl.ds(0, K)], bufs.at[b], ssem.at[b]
            ).wait()
        plsc.subcore_barrier()
        rows = pl.ds(s * ROWS_PER_TILE, ROWS_PER_TILE)

        @pl.when(c == 0)
        def _():
            pltpu.sync_copy(accum.at[rows], oq[p].at[rows])

        @pl.when(c == 1)
        def _():
            pltpu.sync_copy(accum.at[rows], oq[2 + p].at[rows])


RB = 1024


def _mm_body(deg_ref, x_ref, w_ref, g0_ref, g1_ref, g2_ref, g3_ref):
    deg = jnp.sum(deg_ref[...], axis=0)
    dinv = lax.rsqrt(deg)
    h = jnp.dot(x_ref[...], w_ref[...], preferred_element_type=jnp.float32)
    g = (h * dinv[:, None]).astype(jnp.bfloat16)
    g0_ref[...] = g[:, 0 * DQ:1 * DQ]
    g1_ref[...] = g[:, 1 * DQ:2 * DQ]
    g2_ref[...] = g[:, 2 * DQ:3 * DQ]
    g3_ref[...] = g[:, 3 * DQ:4 * DQ]


def _mm(deg, x, w):
    nb = NPAD // RB
    qspec = pl.BlockSpec((RB, DQ), lambda i: (i, 0))
    return pl.pallas_call(
        _mm_body,
        grid=(nb,),
        in_specs=[
            pl.BlockSpec((NW, RB), lambda i: (0, i)),
            pl.BlockSpec((RB, D), lambda i: (i, 0)),
            pl.BlockSpec((D, D), lambda i: (0, 0)),
        ],
        out_specs=(qspec, qspec, qspec, qspec),
        out_shape=(_QSD, _QSD, _QSD, _QSD),
    )(deg, x, w)


def _fin_body(q0_ref, q1_ref, q2_ref, q3_ref, deg_ref, b_ref, o_ref):
    deg = jnp.sum(deg_ref[...], axis=0)
    dinv = lax.rsqrt(deg)
    merged = jnp.concatenate(
        [q0_ref[...], q1_ref[...], q2_ref[...], q3_ref[...]], axis=1
    ).astype(jnp.float32)
    o_ref[...] = merged * dinv[:, None] + b_ref[...][None, :]


def _fin(s0, s1, s2, s3, deg, b):
    nb = NPAD // RB
    qspec = pl.BlockSpec((RB, DQ), lambda i: (i, 0))
    return pl.pallas_call(
        _fin_body,
        grid=(nb,),
        in_specs=[
            qspec,
            qspec,
            qspec,
            qspec,
            pl.BlockSpec((NW, RB), lambda i: (0, i)),
            pl.BlockSpec((D,), lambda i: (0,)),
        ],
        out_specs=pl.BlockSpec((RB, D), lambda i: (i, 0)),
        out_shape=jax.ShapeDtypeStruct((N, D), jnp.float32),
    )(s0, s1, s2, s3, deg, b)


def kernel(x, edge_index, W, b):
    src = edge_index[0].astype(jnp.int32)
    dst = edge_index[1].astype(jnp.int32)
    loop = jnp.arange(N, dtype=jnp.int32)
    npad_e = EPAD - E - N
    srcp = jnp.concatenate([src, loop, jnp.zeros((npad_e,), jnp.int32)])
    dstp = jnp.concatenate([dst, loop, jnp.full((npad_e,), DUMMY, jnp.int32)])
    srcr = srcp.reshape(NSUB, NCH, K)
    dstr = dstp.reshape(NSUB, NCH, K)

    deg = _deg(dstr)
    g0, g1, g2, g3 = _mm(deg, x, W)
    s0, s1, s2, s3 = _prop(srcr, dstr, g0, g1, g2, g3)
    return _fin(s0, s1, s2, s3, deg, b)


# speedup vs baseline: 2.1272x; 1.0982x over previous
import functools

import jax
import jax.numpy as jnp
from jax import lax
from jax.experimental import pallas as pl
from jax.experimental.pallas import tpu as pltpu
from jax.experimental.pallas import tpu_sc as plsc

N = 10000
D = 256
DH = 128
E = 160000
NPAD = 10240
DUMMY = N
NSUB = 16
NCORE = 2
NW = NCORE * NSUB
K = 128
NCH = 84
EPAD = NSUB * NCH * K
NB = 8
ROWS_PER_TILE = NPAD // NSUB

_mesh = plsc.VectorSubcoreMesh(core_axis_name="c", subcore_axis_name="s")


@functools.partial(
    pl.kernel,
    out_type=jax.ShapeDtypeStruct((NW, NPAD), jnp.float32),
    mesh=_mesh,
    compiler_params=pltpu.CompilerParams(needs_layout_passes=False),
    scratch_types=[
        pltpu.VMEM((NCH, K), jnp.int32),
        pltpu.VMEM((NPAD,), jnp.float32),
    ],
)
def _deg(edges_hbm, out_hbm, dstv, ldeg):
    c = lax.axis_index("c")
    s = lax.axis_index("s")
    pltpu.sync_copy(edges_hbm.at[1].at[s], dstv)
    zero16 = jnp.zeros((16,), jnp.float32)
    ones16 = jnp.ones((16,), jnp.float32)

    def zbody(i, _):
        ldeg[pl.ds(i * 16, 16)] = zero16
        return 0

    lax.fori_loop(0, NPAD // 16, zbody, 0)

    half = NCH // 2

    def ebody(j, _):
        jj = c * half + j
        for q in range(K // 16):
            idx = dstv[jj, pl.ds(q * 16, 16)]
            plsc.addupdate_scatter(ldeg, [idx], ones16)
        return 0

    lax.fori_loop(0, half, ebody, 0)
    pltpu.sync_copy(ldeg, out_hbm.at[s * NCORE + c])


_HSD = jax.ShapeDtypeStruct((NPAD, DH), jnp.bfloat16)


@functools.partial(
    pl.kernel,
    out_type=(_HSD, _HSD),
    mesh=_mesh,
    compiler_params=pltpu.CompilerParams(
        needs_layout_passes=False, use_tc_tiling_on_sc=False
    ),
    scratch_types=[
        pltpu.VMEM((NCH, K), jnp.int32),
        pltpu.VMEM((NCH, K), jnp.int32),
        pltpu.VMEM((NB, K, DH), jnp.bfloat16),
        pltpu.VMEM_SHARED((NPAD, DH), jnp.bfloat16),
        pltpu.SemaphoreType.DMA((NB,)),
        pltpu.SemaphoreType.DMA((NB,)),
    ],
)
def _prop(edges_hbm, g0, g1, o0, o1, srcv, dstv, bufs, accum, gsem, ssem):
    c = lax.axis_index("c")
    s = lax.axis_index("s")
    pltpu.sync_copy(edges_hbm.at[0].at[s], srcv)
    pltpu.sync_copy(edges_hbm.at[1].at[s], dstv)

    zero32 = jnp.zeros((32,), jnp.bfloat16)

    def zb(i, _):
        for q in range(DH // 32):
            bufs[0, i, pl.ds(q * 32, 32)] = zero32
        return 0

    lax.fori_loop(0, K, zb, 0)
    for t in range(ROWS_PER_TILE // K):
        pltpu.sync_copy(
            bufs.at[0], accum.at[pl.ds(s * ROWS_PER_TILE + t * K, K)]
        )
    plsc.subcore_barrier()

    def gissue(j, b):
        @pl.when(c == 0)
        def _():
            pltpu.async_copy(g0.at[srcv.at[j]], bufs.at[b], gsem.at[b])

        @pl.when(c == 1)
        def _():
            pltpu.async_copy(g1.at[srcv.at[j]], bufs.at[b], gsem.at[b])

    for j in range(NB):
        gissue(j, j)

    def mb(j, _):
        b = lax.rem(j, NB)
        pltpu.make_async_copy(
            g0.at[pl.ds(0, K)], bufs.at[b], gsem.at[b]
        ).wait()
        pltpu.async_copy(
            bufs.at[b], accum.at[dstv.at[j]], ssem.at[b], add=True
        )
        jn = j + NB

        @pl.when(jn < NCH)
        def _():
            pltpu.make_async_copy(
                g0.at[pl.ds(0, K)], bufs.at[b], ssem.at[b]
            ).wait()
            gissue(jn, b)

        return 0

    lax.fori_loop(0, NCH, mb, 0)
    for b in range(NB):
        pltpu.make_async_copy(
            g0.at[pl.ds(0, K)], bufs.at[b], ssem.at[b]
        ).wait()
    plsc.subcore_barrier()
    rows = pl.ds(s * ROWS_PER_TILE, ROWS_PER_TILE)

    @pl.when(c == 0)
    def _():
        pltpu.sync_copy(accum.at[rows], o0.at[rows])

    @pl.when(c == 1)
    def _():
        pltpu.sync_copy(accum.at[rows], o1.at[rows])


RB = 1024


def _mm_body(deg_ref, x_ref, w_ref, g0_ref, g1_ref):
    deg = jnp.sum(deg_ref[...], axis=0)
    dinv = lax.rsqrt(deg)
    h = jnp.dot(x_ref[...], w_ref[...], preferred_element_type=jnp.float32)
    g = (h * dinv[:, None]).astype(jnp.bfloat16)
    g0_ref[...] = g[:, 0:DH]
    g1_ref[...] = g[:, DH:D]


def _mm(deg, x, w):
    nb = NPAD // RB
    hspec = pl.BlockSpec((RB, DH), lambda i: (i, 0))
    return pl.pallas_call(
        _mm_body,
        grid=(nb,),
        in_specs=[
            pl.BlockSpec((NW, RB), lambda i: (0, i)),
            pl.BlockSpec((RB, D), lambda i: (i, 0)),
            pl.BlockSpec((D, D), lambda i: (0, 0)),
        ],
        out_specs=(hspec, hspec),
        out_shape=(_HSD, _HSD),
    )(deg, x, w)


def _fin_body(h0_ref, h1_ref, deg_ref, b_ref, o_ref):
    deg = jnp.sum(deg_ref[...], axis=0)
    dinv = lax.rsqrt(deg)
    merged = jnp.concatenate(
        [h0_ref[...], h1_ref[...]], axis=1
    ).astype(jnp.float32)
    o_ref[...] = merged * dinv[:, None] + b_ref[...][None, :]


def _fin(s0, s1, deg, b):
    nb = NPAD // RB
    hspec = pl.BlockSpec((RB, DH), lambda i: (i, 0))
    return pl.pallas_call(
        _fin_body,
        grid=(nb,),
        in_specs=[
            hspec,
            hspec,
            pl.BlockSpec((NW, RB), lambda i: (0, i)),
            pl.BlockSpec((D,), lambda i: (0,)),
        ],
        out_specs=pl.BlockSpec((RB, D), lambda i: (i, 0)),
        out_shape=jax.ShapeDtypeStruct((N, D), jnp.float32),
    )(s0, s1, deg, b)


def kernel(x, edge_index, W, b):
    loop = jnp.arange(N, dtype=jnp.int32)
    npad_e = EPAD - E - N
    extra = jnp.stack([
        jnp.concatenate([loop, jnp.zeros((npad_e,), jnp.int32)]),
        jnp.concatenate([loop, jnp.full((npad_e,), DUMMY, jnp.int32)]),
    ])
    edges_p = jnp.concatenate(
        [edge_index.astype(jnp.int32), extra], axis=1
    ).reshape(2, NSUB, NCH, K)

    deg = _deg(edges_p)
    g0, g1 = _mm(deg, x, W)
    s0, s1 = _prop(edges_p, g0, g1)
    return _fin(s0, s1, deg, b)
